# baseline (speedup 1.0000x reference)
import jax
import jax.numpy as jnp
from jax import lax
from jax.experimental import pallas as pl
from jax.experimental.pallas import tpu as pltpu
from jax.experimental.pallas import tpu_sc as plsc

A_DIM = 26
N_CLASSES = 100000
BATCH = 4096

_NC = 2
_NS = 16
_L = 16

_RB = BATCH // 128
_NCHUNK = 5
_CW = N_CLASSES // _NCHUNK
_UNROLL = 10
_CSTEPS = _CW // _L // _UNROLL

_LN2 = 0.6931471805599453


def _vlog(x):
    bits = lax.bitcast_convert_type(x, jnp.int32)
    e = ((bits >> 23) & 0xFF) - 127
    mant = lax.bitcast_convert_type(
        (bits & 0x7FFFFF) | jnp.int32(0x3F800000), jnp.float32
    )
    t = mant - 1.0
    y = e.astype(jnp.float32) * _LN2 + t * (1.0 - t * (0.5 - t * (1.0 / 3.0)))
    for _ in range(3):
        y = y + x * jnp.exp(-y) - 1.0
    return y


def _sc_body(logits_hbm, valt_hbm, out_hbm,
             row_v, col_v, acc_v, sidx_v, shared,
             sem0, sem1, sem2, sem3, sem4):
    c = lax.axis_index("c")
    s = lax.axis_index("s")
    wid = s * _NC + c
    sems = [sem0, sem1, sem2, sem3, sem4]

    @pl.when(wid < A_DIM)
    def _work():
        row0 = pl.multiple_of(wid * N_CLASSES, 8)
        handles = [
            pltpu.async_copy(
                logits_hbm.at[pl.ds(row0 + k * _CW, _CW)],
                row_v.at[pl.ds(k * _CW, _CW)],
                sems[k],
            )
            for k in range(_NCHUNK)
        ]

        pltpu.sync_copy(valt_hbm.at[wid], col_v)

        stats = []
        for k in range(_NCHUNK):
            handles[k].wait()
            base = k * _CW

            def mx(i, carry, base=base):
                a, b = carry
                for u in range(0, _UNROLL, 2):
                    off = base + (i * _UNROLL + u) * _L
                    a = jnp.maximum(a, row_v[pl.ds(off, _L)])
                    b = jnp.maximum(b, row_v[pl.ds(off + _L, _L)])
                return a, b

            ninf = jnp.full((_L,), -jnp.inf, jnp.float32)
            ma, mb = lax.fori_loop(0, _CSTEPS, mx, (ninf, ninf))
            m_k = jnp.max(jnp.maximum(ma, mb))

            def se(i, carry, base=base, m_k=m_k):
                a, b = carry
                for u in range(0, _UNROLL, 2):
                    off = base + (i * _UNROLL + u) * _L
                    a = a + jnp.exp(row_v[pl.ds(off, _L)] - m_k)
                    b = b + jnp.exp(row_v[pl.ds(off + _L, _L)] - m_k)
                return a, b

            zero = jnp.zeros((_L,), jnp.float32)
            sa, sb = lax.fori_loop(0, _CSTEPS, se, (zero, zero))
            stats.append((m_k, jnp.sum(sa + sb)))

        m_tot = stats[0][0]
        for m_k, _ in stats[1:]:
            m_tot = jnp.maximum(m_tot, m_k)
        sacc = jnp.zeros((_L,), jnp.float32)
        for m_k, s_k in stats:
            sacc = sacc + s_k * jnp.exp(jnp.full((_L,), m_k - m_tot, jnp.float32))
        lse_vec = _vlog(sacc) + m_tot

        def outer(r, carry):
            for k in range(128 // _L):
                idx = col_v[pl.ds((r * 8 + k) * _L, _L)]
                acc_v[r, pl.ds(k * _L, _L)] = (
                    plsc.load_gather(row_v, [idx]) - lse_vec
                )
            return carry

        lax.fori_loop(0, _RB, outer, 0)

    iota = lax.broadcasted_iota(jnp.int32, (_L,), 0)
    sidx_v[pl.ds(0, _L)] = iota
    sidx_v[pl.ds(_L, _L)] = iota + _L

    plsc.subcore_barrier()

    @pl.when(s == 0)
    def _seed():
        pltpu.sync_copy(acc_v, shared)

    plsc.subcore_barrier()

    @pl.when((s >= 1) & (wid < A_DIM))
    def _accum():
        pltpu.sync_copy(acc_v, shared.at[sidx_v], add=True)

    plsc.subcore_barrier()

    @pl.when(s == 0)
    def _out():
        pltpu.sync_copy(shared, out_hbm.at[c])


def _sc_gather(logits, valt):
    mesh = plsc.VectorSubcoreMesh(
        core_axis_name="c", subcore_axis_name="s", num_cores=_NC, num_subcores=_NS
    )
    f = pl.kernel(
        _sc_body,
        out_type=jax.ShapeDtypeStruct((_NC, _RB, 128), jnp.float32),
        mesh=mesh,
        scratch_types=[
            pltpu.VMEM((N_CLASSES,), jnp.float32),
            pltpu.VMEM((BATCH,), jnp.int32),
            pltpu.VMEM((_RB, 128), jnp.float32),
            pltpu.VMEM((2 * _L,), jnp.int32),
            pltpu.VMEM_SHARED((_RB, 128), jnp.float32),
            pltpu.SemaphoreType.DMA,
            pltpu.SemaphoreType.DMA,
            pltpu.SemaphoreType.DMA,
            pltpu.SemaphoreType.DMA,
            pltpu.SemaphoreType.DMA,
        ],
        compiler_params=pltpu.CompilerParams(needs_layout_passes=False),
    )
    return f(logits, valt)


def kernel(logits, value):
    valt = value.T
    parts = _sc_gather(logits.reshape(-1), valt)
    return (parts[0] + parts[1]).reshape(BATCH)


# speedup vs baseline: 1.0592x; 1.0592x over previous
import jax
import jax.numpy as jnp
from jax import lax
from jax.experimental import pallas as pl
from jax.experimental.pallas import tpu as pltpu
from jax.experimental.pallas import tpu_sc as plsc

A_DIM = 26
N_CLASSES = 100000
BATCH = 4096

_NC = 2
_NS = 16
_L = 16

_RB = BATCH // 128


def _lse_body(x_ref, o_ref):
    x = x_ref[...]
    m = jnp.max(x, axis=1, keepdims=True)
    s = jnp.sum(jnp.exp(x - m), axis=1, keepdims=True)
    o_ref[0, 0] = jnp.sum(m + jnp.log(s))


def _lse_const(logits):
    return pl.pallas_call(
        _lse_body,
        in_specs=[pl.BlockSpec(memory_space=pltpu.VMEM)],
        out_specs=pl.BlockSpec(memory_space=pltpu.SMEM),
        out_shape=jax.ShapeDtypeStruct((1, 1), jnp.float32),
    )(logits)


def _sc_body(logits_hbm, valt_hbm, out_hbm,
             col_v, idx_v, acc_v, sidx_v, shared, sem):
    c = lax.axis_index("c")
    s = lax.axis_index("s")
    wid = s * _NC + c

    @pl.when(wid < A_DIM)
    def _work():
        pltpu.sync_copy(valt_hbm.at[wid], col_v)
        base = wid * N_CLASSES

        def flatten(r, carry):
            for k in range(128 // _L):
                sl = pl.ds(k * _L, _L)
                idx_v[r, sl] = col_v[r, sl] + base
            return carry

        lax.fori_loop(0, _RB, flatten, 0)

        for g in range(4):
            handles = [
                pltpu.async_copy(
                    logits_hbm.at[idx_v.at[g * 8 + k]],
                    acc_v.at[g * 8 + k],
                    sem,
                )
                for k in range(8)
            ]
            for h in handles:
                h.wait()

    iota = lax.broadcasted_iota(jnp.int32, (_L,), 0)
    sidx_v[pl.ds(0, _L)] = iota
    sidx_v[pl.ds(_L, _L)] = iota + _L

    plsc.subcore_barrier()

    @pl.when(s == 0)
    def _seed():
        pltpu.sync_copy(acc_v, shared)

    plsc.subcore_barrier()

    @pl.when((s >= 1) & (wid < A_DIM))
    def _accum():
        pltpu.sync_copy(acc_v, shared.at[sidx_v], add=True)

    plsc.subcore_barrier()

    @pl.when(s == 0)
    def _out():
        pltpu.sync_copy(shared, out_hbm.at[c])


def _sc_gather(logits_flat, valt):
    mesh = plsc.VectorSubcoreMesh(
        core_axis_name="c", subcore_axis_name="s", num_cores=_NC, num_subcores=_NS
    )
    f = pl.kernel(
        _sc_body,
        out_type=jax.ShapeDtypeStruct((_NC, _RB, 128), jnp.float32),
        mesh=mesh,
        scratch_types=[
            pltpu.VMEM((_RB, 128), jnp.int32),
            pltpu.VMEM((_RB, 128), jnp.int32),
            pltpu.VMEM((_RB, 128), jnp.float32),
            pltpu.VMEM((2 * _L,), jnp.int32),
            pltpu.VMEM_SHARED((_RB, 128), jnp.float32),
            pltpu.SemaphoreType.DMA,
        ],
        compiler_params=pltpu.CompilerParams(needs_layout_passes=False),
    )
    return f(logits_flat, valt)


def kernel(logits, value):
    valt = value.T.reshape(A_DIM, _RB, 128)
    c = _lse_const(logits)
    parts = _sc_gather(logits.reshape(-1), valt)
    return (parts[0] + parts[1]).reshape(BATCH) - c[0, 0]


# speedup vs baseline: 1.0706x; 1.0107x over previous
import jax
import jax.numpy as jnp
from jax import lax
from jax.experimental import pallas as pl
from jax.experimental.pallas import tpu as pltpu
from jax.experimental.pallas import tpu_sc as plsc

A_DIM = 26
N_CLASSES = 100000
BATCH = 4096

_NC = 2
_NS = 16
_L = 16

_RB = BATCH // 128


_W = 8192
_G = -(-N_CLASSES // _W)


def _lse_body(x_ref, o_ref, m_ref, s_ref):
    g = pl.program_id(0)

    @pl.when(g == 0)
    def _init():
        m_ref[...] = jnp.full_like(m_ref, -jnp.inf)
        s_ref[...] = jnp.zeros_like(s_ref)

    x = x_ref[...]
    col = g * _W + lax.broadcasted_iota(jnp.int32, x.shape, 1)
    x = jnp.where(col < N_CLASSES, x, -jnp.inf)
    bm = jnp.max(x, axis=1, keepdims=True)
    m_old = m_ref[...]
    m_new = jnp.maximum(m_old, bm)
    s_new = s_ref[...] * jnp.exp(m_old - m_new) + jnp.sum(
        jnp.exp(x - m_new), axis=1, keepdims=True
    )
    m_ref[...] = m_new
    s_ref[...] = s_new

    @pl.when(g == _G - 1)
    def _fin():
        o_ref[0, 0] = jnp.sum(m_new + jnp.log(s_new))


def _lse_const(logits):
    return pl.pallas_call(
        _lse_body,
        grid=(_G,),
        in_specs=[pl.BlockSpec((A_DIM, _W), lambda g: (0, g))],
        out_specs=pl.BlockSpec(memory_space=pltpu.SMEM),
        out_shape=jax.ShapeDtypeStruct((1, 1), jnp.float32),
        scratch_shapes=[
            pltpu.VMEM((A_DIM, 1), jnp.float32),
            pltpu.VMEM((A_DIM, 1), jnp.float32),
        ],
    )(logits)


def _sc_body(logits_hbm, valt_hbm, out_hbm,
             col_v, gth_v, acc_v, sidx_v, shared, sem):
    c = lax.axis_index("c")
    s = lax.axis_index("s")
    wid = s * _NC + c

    @pl.when(wid < A_DIM)
    def _work():
        pltpu.sync_copy(valt_hbm.at[wid], col_v)
        row0 = pl.multiple_of(wid * N_CLASSES, 8)
        pltpu.async_copy(
            logits_hbm.at[pl.ds(row0, N_CLASSES)].at[col_v], gth_v, sem
        ).wait()

        def reshape(r, carry):
            for k in range(128 // _L):
                acc_v[r, pl.ds(k * _L, _L)] = gth_v[pl.ds((r * 8 + k) * _L, _L)]
            return carry

        lax.fori_loop(0, _RB, reshape, 0)

    iota = lax.broadcasted_iota(jnp.int32, (_L,), 0)
    sidx_v[pl.ds(0, _L)] = iota
    sidx_v[pl.ds(_L, _L)] = iota + _L

    plsc.subcore_barrier()

    @pl.when(s == 0)
    def _seed():
        pltpu.sync_copy(acc_v, shared)

    plsc.subcore_barrier()

    @pl.when((s >= 1) & (wid < A_DIM))
    def _accum():
        pltpu.sync_copy(acc_v, shared.at[sidx_v], add=True)

    plsc.subcore_barrier()

    @pl.when(s == 0)
    def _out():
        pltpu.sync_copy(shared, out_hbm.at[c])


def _sc_gather(logits_flat, valt):
    mesh = plsc.VectorSubcoreMesh(
        core_axis_name="c", subcore_axis_name="s", num_cores=_NC, num_subcores=_NS
    )
    f = pl.kernel(
        _sc_body,
        out_type=jax.ShapeDtypeStruct((_NC, _RB, 128), jnp.float32),
        mesh=mesh,
        scratch_types=[
            pltpu.VMEM((BATCH,), jnp.int32),
            pltpu.VMEM((BATCH,), jnp.float32),
            pltpu.VMEM((_RB, 128), jnp.float32),
            pltpu.VMEM((2 * _L,), jnp.int32),
            pltpu.VMEM_SHARED((_RB, 128), jnp.float32),
            pltpu.SemaphoreType.DMA,
        ],
        compiler_params=pltpu.CompilerParams(needs_layout_passes=False),
    )
    return f(logits_flat, valt)


def kernel(logits, value):
    valt = value.T
    c = _lse_const(logits)
    parts = _sc_gather(logits.reshape(-1), valt)
    return (parts[0] + parts[1]).reshape(BATCH) - c[0, 0]


# speedup vs baseline: 1.4679x; 1.3711x over previous
import jax
import jax.numpy as jnp
from jax import lax
from jax.experimental import pallas as pl
from jax.experimental.pallas import tpu as pltpu
from jax.experimental.pallas import tpu_sc as plsc

A_DIM = 26
N_CLASSES = 100000
BATCH = 4096

_NC = 2
_NS = 16
_L = 16

_RB = BATCH // 128


_W = 16384
_G = -(-N_CLASSES // _W)


def _lse_body(x_ref, o_ref, m_ref, s_ref):
    g = pl.program_id(0)

    @pl.when(g == 0)
    def _init():
        m_ref[...] = jnp.full_like(m_ref, -jnp.inf)
        s_ref[...] = jnp.zeros_like(s_ref)

    x = x_ref[...]

    def _masked():
        col = g * _W + lax.broadcasted_iota(jnp.int32, x.shape, 1)
        return jnp.where(col < N_CLASSES, x, -jnp.inf)

    x = lax.cond(g == _G - 1, _masked, lambda: x)
    bm = jnp.max(x, axis=1, keepdims=True)
    m_old = m_ref[...]
    m_new = jnp.maximum(m_old, bm)
    s_new = s_ref[...] * jnp.exp(m_old - m_new) + jnp.sum(
        jnp.exp(x - m_new), axis=1, keepdims=True
    )
    m_ref[...] = m_new
    s_ref[...] = s_new

    @pl.when(g == _G - 1)
    def _fin():
        o_ref[0, 0] = jnp.sum(m_new + jnp.log(s_new))


def _lse_const(logits):
    return pl.pallas_call(
        _lse_body,
        grid=(_G,),
        in_specs=[pl.BlockSpec((A_DIM, _W), lambda g: (0, g))],
        out_specs=pl.BlockSpec(memory_space=pltpu.SMEM),
        out_shape=jax.ShapeDtypeStruct((1, 1), jnp.float32),
        scratch_shapes=[
            pltpu.VMEM((A_DIM, 1), jnp.float32),
            pltpu.VMEM((A_DIM, 1), jnp.float32),
        ],
    )(logits)


def _sc_body(logits_hbm, valt_hbm, out_hbm, row_v, idx_v, acc_v, sidx_v, shared):
    c = lax.axis_index("c")
    s = lax.axis_index("s")
    wid = s * _NC + c

    @pl.when(wid < A_DIM)
    def _work():
        pltpu.sync_copy(logits_hbm.at[wid], row_v)
        pltpu.sync_copy(valt_hbm.at[wid], idx_v)

        def outer(r, carry):
            for k in range(128 // _L):
                sl = pl.ds(k * _L, _L)
                acc_v[r, sl] = plsc.load_gather(row_v, [idx_v[r, sl]])
            return carry

        lax.fori_loop(0, _RB, outer, 0)

    iota = lax.broadcasted_iota(jnp.int32, (_L,), 0)
    sidx_v[pl.ds(0, _L)] = iota
    sidx_v[pl.ds(_L, _L)] = iota + _L

    plsc.subcore_barrier()

    @pl.when(s == 0)
    def _seed():
        pltpu.sync_copy(acc_v, shared)

    plsc.subcore_barrier()

    @pl.when((s >= 1) & (wid < A_DIM))
    def _accum():
        pltpu.sync_copy(acc_v, shared.at[sidx_v], add=True)

    plsc.subcore_barrier()

    @pl.when(s == 0)
    def _out():
        pltpu.sync_copy(shared, out_hbm.at[c])


def _sc_gather(logits, valt):
    mesh = plsc.VectorSubcoreMesh(
        core_axis_name="c", subcore_axis_name="s", num_cores=_NC, num_subcores=_NS
    )
    f = pl.kernel(
        _sc_body,
        out_type=jax.ShapeDtypeStruct((_NC, _RB, 128), jnp.float32),
        mesh=mesh,
        scratch_types=[
            pltpu.VMEM((N_CLASSES,), jnp.float32),
            pltpu.VMEM((_RB, 128), jnp.int32),
            pltpu.VMEM((_RB, 128), jnp.float32),
            pltpu.VMEM((2 * _L,), jnp.int32),
            pltpu.VMEM_SHARED((_RB, 128), jnp.float32),
        ],
        compiler_params=pltpu.CompilerParams(needs_layout_passes=False),
    )
    return f(logits, valt)


def kernel(logits, value):
    valt = value.T.reshape(A_DIM, _RB, 128)
    parts = _sc_gather(logits, valt)
    c = _lse_const(logits)
    return (parts[0] + parts[1]).reshape(BATCH) - c[0, 0]
